# Initial kernel scaffold; baseline (speedup 1.0000x reference)
import jax
import jax.numpy as jnp
from jax.experimental import pallas as pl


def kernel(node_feat, total_edge, total_relation, total_relation_embed, total_target_relation, source_node, target_node, graph_sizes, params):
    raise NotImplementedError("write your pallas kernel here")


# speedup vs baseline: 1.0195x; 1.0195x over previous
import functools

import jax
import jax.numpy as jnp
from jax import lax
from jax.experimental import pallas as pl
from jax.experimental.pallas import tpu as pltpu
from jax.experimental.pallas import tpu_sc as plsc

N = 10000
E = 160000
D = 128
R = 16
NB = 4
A_DIM = 32
G = 100

NC = 2
NS = 16
NW = NC * NS
C = 64
NCHUNK = E // C
KMAX = (NCHUNK + NW - 1) // NW
N_PAD = 10240
AGG_R = 5120
SLAB = AGG_R // NS
DSTEP = 64

_BN = 2000
_BE = 4000


def _node_mm_body(h_ref, w_ref, tsrc_ref, curr_ref, ptgt_ref):
    y = jnp.dot(h_ref[...], w_ref[...], preferred_element_type=jnp.float32)
    tsrc_ref[...] = y[:, : NB * D + D]
    curr_ref[...] = y[:, NB * D + D : NB * D + 2 * D]
    ptgt_ref[...] = y[:, NB * D + 2 * D :]


def _edge_feat_body(re_ref, tr_ref, rel_ref, wre_ref, wtr_ref, ab_ref,
                    comp_ref, eb_ref, co1_ref, co2_ref):
    eb_ref[...] = (
        jnp.dot(re_ref[...], wre_ref[...], preferred_element_type=jnp.float32)
        + jnp.dot(tr_ref[...], wtr_ref[...], preferred_element_type=jnp.float32)
        + ab_ref[...])
    onehot = (rel_ref[...] == lax.broadcasted_iota(jnp.int32, (1, R), 1)
              ).astype(jnp.float32)
    co = jnp.dot(onehot, comp_ref[...], preferred_element_type=jnp.float32)
    co1_ref[...] = co[:, :16]
    co2_ref[...] = co[:, 16:]


def _combine_body(curr_ref, p0_ref, p1_ref, h_ref):
    h_ref[...] = jnp.maximum(curr_ref[...] + p0_ref[...] + p1_ref[...], 0.0)


def _final_body(h1_ref, h2_ref, total_ref):
    total_ref[:, :D] = h1_ref[...]
    total_ref[:, D:] = h2_ref[...]


def _pool_body(gszf_ref, total_ref, ge_ref):
    t = total_ref[...].reshape(G, N // G, 2 * D)
    ge_ref[...] = jnp.sum(t, axis=1) / gszf_ref[...]


def _make_edge_sc():
    mesh = plsc.VectorSubcoreMesh(core_axis_name="c", subcore_axis_name="s",
                                  num_cores=NC, num_subcores=NS)

    @functools.partial(
        pl.kernel,
        out_type=[jax.ShapeDtypeStruct((NC, N_PAD, D), jnp.float32),
                  jax.ShapeDtypeStruct((E, D), jnp.float32)],
        mesh=mesh,
        scratch_types=[
            pltpu.VMEM((C,), jnp.int32),
            pltpu.VMEM((C,), jnp.int32),
            pltpu.VMEM((C,), jnp.int32),
            pltpu.VMEM((C, NB * D + D), jnp.float32),
            pltpu.VMEM((C, D), jnp.float32),
            pltpu.VMEM((C, D), jnp.float32),
            pltpu.VMEM((C, 16), jnp.float32),
            pltpu.VMEM((C, D), jnp.float32),
            pltpu.VMEM((D + 16,), jnp.float32),
            pltpu.VMEM((16,), jnp.int32),
            pltpu.VMEM((DSTEP, D), jnp.float32),
            pltpu.VMEM_SHARED((AGG_R + 8, D), jnp.float32),
            pltpu.SemaphoreType.DMA,
            pltpu.SemaphoreType.DMA,
        ],
    )
    def edge_kernel(tsrc_hbm, ptgt_hbm, eb_hbm, co_hbm, src_hbm, tgt_hbm,
                    bw_hbm, lidx_hbm, out_hbm, spill_hbm, src_v, tgt_v, tgt2_v,
                    g_v, pt_v, eb_v, co_v, out_v, bw_v, lidx_v, zbuf_v, agg_sh,
                    sem1, sem2):
        cid = lax.axis_index("c")
        sid = lax.axis_index("s")
        wid = sid * NC + cid
        row0 = sid * SLAB

        def zero_fill(i, _):
            zbuf_v[i // (D // 16), pl.ds((i % (D // 16)) * 16, 16)] = (
                jnp.zeros((16,), jnp.float32))
            return 0
        lax.fori_loop(0, DSTEP * (D // 16), zero_fill, 0)

        def zero_agg():
            for t in range(SLAB // DSTEP):
                pltpu.sync_copy(zbuf_v,
                                agg_sh.at[pl.ds(row0 + t * DSTEP, DSTEP)])

            @pl.when(sid == 0)
            def _():
                pltpu.sync_copy(zbuf_v.at[pl.ds(0, 8)],
                                agg_sh.at[pl.ds(AGG_R, 8)])

        def drain_agg(lo):
            for t in range(SLAB // DSTEP):
                r = row0 + t * DSTEP
                pltpu.sync_copy(agg_sh.at[pl.ds(r, DSTEP)], out_v)
                pltpu.sync_copy(out_v, out_hbm.at[cid, pl.ds(lo + r, DSTEP)])

        def clamp_tgt(lo):
            for q in range(C // 16):
                sl = pl.ds(q * 16, 16)
                t2 = tgt_v[sl] - lo
                ok = (t2 >= 0) & (t2 < AGG_R)
                tgt2_v[sl] = jnp.where(ok, t2, AGG_R)

        zero_agg()
        pltpu.sync_copy(bw_hbm, bw_v)
        pltpu.sync_copy(lidx_hbm, lidx_v)
        eb_col = lidx_v[...][0] * D
        plsc.subcore_barrier()

        def chunk_body(k, _):
            j = wid + NW * k

            @pl.when(j < NCHUNK)
            def _():
                base = j * C
                pltpu.sync_copy(src_hbm.at[pl.ds(base, C)], src_v)
                pltpu.sync_copy(tgt_hbm.at[pl.ds(base, C)], tgt_v)
                cp1 = pltpu.async_copy(tsrc_hbm.at[src_v], g_v, sem1)
                cp2 = pltpu.async_copy(ptgt_hbm.at[tgt_v], pt_v, sem2)
                pltpu.sync_copy(
                    eb_hbm.at[pl.ds(base, C), pl.ds(eb_col, D)], eb_v)
                pltpu.sync_copy(co_hbm.at[pl.ds(base, C)], co_v)
                cp1.wait()
                cp2.wait()

                def edge_body(e, _):
                    acc = bw_v[pl.ds(D, 16)]
                    for g in range(D // 16):
                        sl = pl.ds(g * 16, 16)
                        hid = (g_v[e, pl.ds(NB * D + g * 16, 16)]
                               + pt_v[e, sl] + eb_v[e, sl])
                        acc = acc + jnp.maximum(hid, 0.0) * bw_v[sl]
                    for sh in (8, 4, 2, 1):
                        perm = jnp.arange(16, dtype=jnp.int32) ^ sh
                        acc = acc + lax.gather(
                            acc, perm[:, None],
                            lax.GatherDimensionNumbers(
                                offset_dims=(), collapsed_slice_dims=(0,),
                                start_index_map=(0,)),
                            (1,),
                            mode=lax.GatherScatterMode.PROMISE_IN_BOUNDS)
                    av = 1.0 / (1.0 + jnp.exp(-acc))
                    cvec = co_v[e, pl.ds(0, 16)]
                    c0 = jnp.full((16,), cvec[0], jnp.float32)
                    c1 = jnp.full((16,), cvec[1], jnp.float32)
                    c2 = jnp.full((16,), cvec[2], jnp.float32)
                    c3 = jnp.full((16,), cvec[3], jnp.float32)
                    for g in range(D // 16):
                        sl = pl.ds(g * 16, 16)
                        m = (c0 * g_v[e, sl]
                             + c1 * g_v[e, pl.ds(D + g * 16, 16)]
                             + c2 * g_v[e, pl.ds(2 * D + g * 16, 16)]
                             + c3 * g_v[e, pl.ds(3 * D + g * 16, 16)])
                        out_v[e, sl] = av * m
                    return 0

                lax.fori_loop(0, C, edge_body, 0)
                pltpu.sync_copy(out_v, spill_hbm.at[pl.ds(base, C)])
                clamp_tgt(0)
                pltpu.sync_copy(out_v, agg_sh.at[tgt2_v], add=True)
            return 0

        lax.fori_loop(0, KMAX, chunk_body, 0)
        plsc.subcore_barrier()
        drain_agg(0)
        zero_agg()
        plsc.subcore_barrier()

        def fix_body(k, _):
            j = wid + NW * k

            @pl.when(j < NCHUNK)
            def _():
                base = j * C
                pltpu.sync_copy(tgt_hbm.at[pl.ds(base, C)], tgt_v)
                pltpu.sync_copy(spill_hbm.at[pl.ds(base, C)], out_v)
                clamp_tgt(AGG_R)
                pltpu.sync_copy(out_v, agg_sh.at[tgt2_v], add=True)
            return 0

        lax.fori_loop(0, KMAX, fix_body, 0)
        plsc.subcore_barrier()
        drain_agg(AGG_R)

    return edge_kernel


def _make_embed_gather():
    mesh = plsc.VectorSubcoreMesh(core_axis_name="c", subcore_axis_name="s",
                                  num_cores=NC, num_subcores=NS)

    @functools.partial(
        pl.kernel,
        out_type=jax.ShapeDtypeStruct((2 * 128, 2 * D), jnp.float32),
        mesh=mesh,
        scratch_types=[
            pltpu.VMEM((128,), jnp.int32),
            pltpu.VMEM((128, 2 * D), jnp.float32),
            pltpu.SemaphoreType.DMA,
        ],
    )
    def gather_kernel(total_hbm, idx_hbm, out_hbm, idx_v, rows_v, sem):
        wid = lax.axis_index("s") * NC + lax.axis_index("c")

        @pl.when(wid < 2)
        def _():
            base = wid * 128
            pltpu.sync_copy(idx_hbm.at[pl.ds(base, 128)], idx_v)
            pltpu.async_copy(total_hbm.at[idx_v], rows_v, sem).wait()
            pltpu.sync_copy(rows_v, out_hbm.at[pl.ds(base, 128)])

    return gather_kernel


_get_edge_sc = functools.lru_cache(maxsize=None)(_make_edge_sc)
_get_embed_gather = functools.lru_cache(maxsize=None)(_make_embed_gather)


def _node_mm(h, wcat):
    return pl.pallas_call(
        _node_mm_body,
        grid=(N // _BN,),
        in_specs=[
            pl.BlockSpec((_BN, D), lambda i: (i, 0)),
            pl.BlockSpec((D, 7 * D), lambda i: (0, 0)),
        ],
        out_specs=[
            pl.BlockSpec((_BN, 5 * D), lambda i: (i, 0)),
            pl.BlockSpec((_BN, D), lambda i: (i, 0)),
            pl.BlockSpec((_BN, D), lambda i: (i, 0)),
        ],
        out_shape=[
            jax.ShapeDtypeStruct((N, 5 * D), jnp.float32),
            jax.ShapeDtypeStruct((N, D), jnp.float32),
            jax.ShapeDtypeStruct((N, D), jnp.float32),
        ],
    )(h, wcat)


def _edge_feat(re, tr, rel2d, wre, wtr, ab, comp_cat):
    return pl.pallas_call(
        _edge_feat_body,
        grid=(E // _BE,),
        in_specs=[
            pl.BlockSpec((_BE, A_DIM), lambda i: (i, 0)),
            pl.BlockSpec((_BE, A_DIM), lambda i: (i, 0)),
            pl.BlockSpec((_BE, 1), lambda i: (i, 0)),
            pl.BlockSpec((A_DIM, 2 * D), lambda i: (0, 0)),
            pl.BlockSpec((A_DIM, 2 * D), lambda i: (0, 0)),
            pl.BlockSpec((1, 2 * D), lambda i: (0, 0)),
            pl.BlockSpec((R, 32), lambda i: (0, 0)),
        ],
        out_specs=[
            pl.BlockSpec((_BE, 2 * D), lambda i: (i, 0)),
            pl.BlockSpec((_BE, 16), lambda i: (i, 0)),
            pl.BlockSpec((_BE, 16), lambda i: (i, 0)),
        ],
        out_shape=[
            jax.ShapeDtypeStruct((E, 2 * D), jnp.float32),
            jax.ShapeDtypeStruct((E, 16), jnp.float32),
            jax.ShapeDtypeStruct((E, 16), jnp.float32),
        ],
    )(re, tr, rel2d, wre, wtr, ab, comp_cat)


def _combine(curr, p0, p1):
    return pl.pallas_call(
        _combine_body,
        grid=(N // _BN,),
        in_specs=[pl.BlockSpec((_BN, D), lambda i: (i, 0))] * 3,
        out_specs=pl.BlockSpec((_BN, D), lambda i: (i, 0)),
        out_shape=jax.ShapeDtypeStruct((N, D), jnp.float32),
    )(curr, p0, p1)


def _final(h1, h2):
    return pl.pallas_call(
        _final_body,
        grid=(N // _BN,),
        in_specs=[pl.BlockSpec((_BN, D), lambda i: (i, 0))] * 2,
        out_specs=pl.BlockSpec((_BN, 2 * D), lambda i: (i, 0)),
        out_shape=jax.ShapeDtypeStruct((N, 2 * D), jnp.float32),
    )(h1, h2)


def _pool(gszf, total):
    return pl.pallas_call(
        _pool_body,
        out_shape=jax.ShapeDtypeStruct((G, 2 * D), jnp.float32),
    )(gszf, total)


def kernel(node_feat, total_edge, total_relation, total_relation_embed,
           total_target_relation, source_node, target_node, graph_sizes,
           params):
    (Wb1, comp1, sloop1, Aw1, Ab1, Bw1, Bb1) = params[0]
    (Wb2, comp2, sloop2, Aw2, Ab2, Bw2, Bb2) = params[1]

    def build_wcat(Wb, sloop, Aw):
        wb_flat = jnp.transpose(Wb, (1, 0, 2)).reshape(D, NB * D)
        return jnp.concatenate(
            [wb_flat, Aw[:, :D].T, sloop, Aw[:, D:2 * D].T], axis=1)

    wcat = jnp.stack([build_wcat(Wb1, sloop1, Aw1),
                      build_wcat(Wb2, sloop2, Aw2)])
    wre = jnp.concatenate(
        [Aw1[:, 2 * D:2 * D + A_DIM].T, Aw2[:, 2 * D:2 * D + A_DIM].T], axis=1)
    wtr = jnp.concatenate(
        [Aw1[:, 2 * D + A_DIM:].T, Aw2[:, 2 * D + A_DIM:].T], axis=1)
    ab = jnp.concatenate([Ab1, Ab2])[None, :]
    comp_cat = jnp.concatenate(
        [comp1, jnp.zeros((R, 12), jnp.float32),
         comp2, jnp.zeros((R, 12), jnp.float32)], axis=1)
    bwbb = jnp.stack([
        jnp.concatenate([Bw1[0], jnp.full((16,), Bb1[0] / 16.0)]),
        jnp.concatenate([Bw2[0], jnp.full((16,), Bb2[0] / 16.0)]),
    ])
    lidx = jnp.stack([jnp.zeros((16,), jnp.int32), jnp.ones((16,), jnp.int32)])

    src = total_edge[0].astype(jnp.int32)
    tgt = total_edge[1].astype(jnp.int32)
    rel2d = total_relation.astype(jnp.int32).reshape(E, 1)
    idx_pad = jnp.concatenate([
        jnp.pad(source_node.astype(jnp.int32), (0, 128 - G)),
        jnp.pad(target_node.astype(jnp.int32), (0, 128 - G)),
    ])
    gszf = graph_sizes.astype(jnp.float32).reshape(G, 1)

    eb_cat, co1, co2 = _edge_feat(total_relation_embed, total_target_relation,
                                  rel2d, wre, wtr, ab, comp_cat)
    co_s = jnp.stack([co1, co2])
    edge_sc = _get_edge_sc()

    def layer_body(h, xs):
        wcat_l, co_l, bwbb_l, lidx_l = xs
        tsrc, curr, ptgt = _node_mm(h, wcat_l)
        parts, _ = edge_sc(tsrc, ptgt, eb_cat, co_l, src, tgt, bwbb_l, lidx_l)
        h_next = _combine(curr, parts[0, :N], parts[1, :N])
        return h_next, h_next

    _, hs = lax.scan(layer_body, node_feat, (wcat, co_s, bwbb, lidx))

    total = _final(hs[0], hs[1])
    graph_embed = _pool(gszf, total)
    sg = _get_embed_gather()(total, idx_pad)
    return (graph_embed, sg[:G], sg[128:128 + G])


# speedup vs baseline: 1.1639x; 1.1417x over previous
import functools

import jax
import jax.numpy as jnp
from jax import lax
from jax.experimental import pallas as pl
from jax.experimental.pallas import tpu as pltpu
from jax.experimental.pallas import tpu_sc as plsc

N = 10000
E = 160000
D = 128
R = 16
NB = 4
A_DIM = 32
G = 100

NC = 2
NS = 16
NW = NC * NS
C = 32
NCHUNK = E // C
KMAX = (NCHUNK + NW - 1) // NW
N_PAD = 10240
AGG_R = 5120
DUMP_R = 1024
AGG_D = AGG_R + DUMP_R
SLAB = AGG_R // NS

_BN = 2000
_BE = 4000


def _node_mm_body(h_ref, w_ref, tsrc_ref, curr_ref, ptgt_ref):
    y = jnp.dot(h_ref[...], w_ref[...], preferred_element_type=jnp.float32)
    tsrc_ref[...] = y[:, : NB * D + D]
    curr_ref[...] = y[:, NB * D + D : NB * D + 2 * D]
    ptgt_ref[...] = y[:, NB * D + 2 * D :]


def _edge_feat_body(re_ref, tr_ref, rel_ref, wre_ref, wtr_ref, ab_ref,
                    comp_ref, eb_ref, co1_ref, co2_ref):
    eb_ref[...] = (
        jnp.dot(re_ref[...], wre_ref[...], preferred_element_type=jnp.float32)
        + jnp.dot(tr_ref[...], wtr_ref[...], preferred_element_type=jnp.float32)
        + ab_ref[...])
    onehot = (rel_ref[...] == lax.broadcasted_iota(jnp.int32, (1, R), 1)
              ).astype(jnp.float32)
    co = jnp.dot(onehot, comp_ref[...], preferred_element_type=jnp.float32)
    co1_ref[...] = co[:, :4]
    co2_ref[...] = co[:, 4:]


def _combine_body(curr_ref, p0_ref, p1_ref, h_ref):
    h_ref[...] = jnp.maximum(curr_ref[...] + p0_ref[0] + p1_ref[0], 0.0)


def _final_body(h1_ref, h2_ref, total_ref):
    total_ref[:, :D] = h1_ref[...]
    total_ref[:, D:] = h2_ref[...]


def _pool_body(gszf_ref, total_ref, ge_ref):
    t = total_ref[...].reshape(G, N // G, 2 * D)
    ge_ref[...] = jnp.sum(t, axis=1) / gszf_ref[...]


def _make_edge_sc():
    mesh = plsc.VectorSubcoreMesh(core_axis_name="c", subcore_axis_name="s",
                                  num_cores=NC, num_subcores=NS)

    @functools.partial(
        pl.kernel,
        out_type=[jax.ShapeDtypeStruct((NC, N_PAD, D), jnp.float32),
                  jax.ShapeDtypeStruct((E, D), jnp.float32)],
        mesh=mesh,
        scratch_types=[
            pltpu.VMEM((C,), jnp.int32),
            pltpu.VMEM((C,), jnp.int32),
            pltpu.VMEM((C,), jnp.int32),
            pltpu.VMEM((C,), jnp.int32),
            pltpu.VMEM((C,), jnp.int32),
            pltpu.VMEM((C, NB * D + D), jnp.float32),
            pltpu.VMEM((C, NB * D + D), jnp.float32),
            pltpu.VMEM((C, D), jnp.float32),
            pltpu.VMEM((C, D), jnp.float32),
            pltpu.VMEM((C, D), jnp.float32),
            pltpu.VMEM((C, D), jnp.float32),
            pltpu.VMEM((C * 4,), jnp.float32),
            pltpu.VMEM((C * 4,), jnp.float32),
            pltpu.VMEM((C, D), jnp.float32),
            pltpu.VMEM((D + 16,), jnp.float32),
            pltpu.VMEM((16,), jnp.int32),
            pltpu.VMEM((32, D), jnp.float32),
            pltpu.VMEM_SHARED((AGG_D, D), jnp.float32),
            pltpu.SemaphoreType.DMA,
            pltpu.SemaphoreType.DMA,
        ],
    )
    def edge_kernel(tsrc_hbm, ptgt_hbm, eb_hbm, co_hbm, src_hbm, tgt_hbm,
                    bw_hbm, lidx_hbm, out_hbm, spill_hbm, src0_v, src1_v,
                    tgt0_v, tgt1_v, tgt2_v, g0_v, g1_v, pt0_v, pt1_v, eb0_v,
                    eb1_v, co0_v, co1_v, out_v, bw_v, lidx_v, zbuf_v, agg_sh,
                    semA, semB):
        cid = lax.axis_index("c")
        sid = lax.axis_index("s")
        wid = sid * NC + cid
        row0 = sid * SLAB
        bufs = ((src0_v, tgt0_v, g0_v, pt0_v, eb0_v, co0_v, semA),
                (src1_v, tgt1_v, g1_v, pt1_v, eb1_v, co1_v, semB))

        def zero_fill(i, _):
            zbuf_v[i // (D // 16), pl.ds((i % (D // 16)) * 16, 16)] = (
                jnp.zeros((16,), jnp.float32))
            return 0
        lax.fori_loop(0, 32 * (D // 16), zero_fill, 0)

        def zero_agg():
            for t in range(SLAB // 32):
                pltpu.sync_copy(zbuf_v, agg_sh.at[pl.ds(row0 + t * 32, 32)])

        def drain_agg(lo):
            for t in range(SLAB // C):
                r = row0 + t * C
                pltpu.sync_copy(agg_sh.at[pl.ds(r, C)], out_v)
                pltpu.sync_copy(out_v, out_hbm.at[cid, pl.ds(lo + r, C)])

        def clamp_scatter(tgtb, lo, rows):
            for q in range(C // 16):
                sl = pl.ds(q * 16, 16)
                t2 = tgtb[sl] - lo
                ok = (t2 >= 0) & (t2 < AGG_R)
                tgt2_v[sl] = jnp.where(ok, t2, AGG_R + (t2 & (DUMP_R - 1)))
            pltpu.sync_copy(rows, agg_sh.at[tgt2_v], add=True)

        zero_agg()
        pltpu.sync_copy(bw_hbm, bw_v)
        pltpu.sync_copy(lidx_hbm, lidx_v)
        eb_col = lidx_v[...][0] * D
        plsc.subcore_barrier()

        def prefetch(j, b):
            srcb, tgtb, gb, ptb, ebb, cob, sem = bufs[b]
            base = j * C
            pltpu.sync_copy(src_hbm.at[pl.ds(base, C)], srcb)
            pltpu.sync_copy(tgt_hbm.at[pl.ds(base, C)], tgtb)
            pltpu.async_copy(tsrc_hbm.at[srcb], gb, sem)
            pltpu.async_copy(ptgt_hbm.at[tgtb], ptb, sem)
            pltpu.async_copy(eb_hbm.at[pl.ds(base, C), pl.ds(eb_col, D)],
                             ebb, sem)
            pltpu.async_copy(co_hbm.at[pl.ds(base * 4, C * 4)], cob, sem)

        def wait4(j, b):
            srcb, tgtb, gb, ptb, ebb, cob, sem = bufs[b]
            base = j * C
            pltpu.make_async_copy(tsrc_hbm.at[srcb], gb, sem).wait()
            pltpu.make_async_copy(ptgt_hbm.at[tgtb], ptb, sem).wait()
            pltpu.make_async_copy(
                eb_hbm.at[pl.ds(base, C), pl.ds(eb_col, D)], ebb, sem).wait()
            pltpu.make_async_copy(
                co_hbm.at[pl.ds(base * 4, C * 4)], cob, sem).wait()

        def compute_chunk(j, b):
            srcb, tgtb, gb, ptb, ebb, cob, sem = bufs[b]
            base = j * C
            bws = tuple(bw_v[pl.ds(g * 16, 16)] for g in range(D // 16))
            acc0 = bw_v[pl.ds(D, 16)]

            def octet_body(q, _):
                cva = cob[pl.ds(q * 32, 16)]
                cvb = cob[pl.ds(q * 32 + 16, 16)]
                for i in range(8):
                    e = q * 8 + i
                    cvec = cva if i < 4 else cvb
                    ci = (4 * i) % 16
                    acc = acc0
                    for g in range(D // 16):
                        sl = pl.ds(g * 16, 16)
                        hid = (gb[e, pl.ds(NB * D + g * 16, 16)]
                               + ptb[e, sl] + ebb[e, sl])
                        acc = acc + jnp.maximum(hid, 0.0) * bws[g]
                    for sh in (8, 4, 2, 1):
                        perm = jnp.arange(16, dtype=jnp.int32) ^ sh
                        acc = acc + lax.gather(
                            acc, perm[:, None],
                            lax.GatherDimensionNumbers(
                                offset_dims=(), collapsed_slice_dims=(0,),
                                start_index_map=(0,)),
                            (1,),
                            mode=lax.GatherScatterMode.PROMISE_IN_BOUNDS)
                    av = 1.0 / (1.0 + jnp.exp(-acc))
                    z0 = av * cvec[ci]
                    z1 = av * cvec[ci + 1]
                    z2 = av * cvec[ci + 2]
                    z3 = av * cvec[ci + 3]
                    for g in range(D // 16):
                        sl = pl.ds(g * 16, 16)
                        out_v[e, sl] = (
                            z0 * gb[e, sl]
                            + z1 * gb[e, pl.ds(D + g * 16, 16)]
                            + z2 * gb[e, pl.ds(2 * D + g * 16, 16)]
                            + z3 * gb[e, pl.ds(3 * D + g * 16, 16)])
                return 0

            lax.fori_loop(0, C // 8, octet_body, 0)
            pltpu.sync_copy(out_v, spill_hbm.at[pl.ds(base, C)])
            clamp_scatter(tgtb, 0, out_v)

        j0 = wid

        @pl.when(j0 < NCHUNK)
        def _():
            prefetch(j0, 0)

        def super_body(m, _):
            for par in range(2):
                k = 2 * m + par
                j = wid + NW * k
                jn = j + NW

                @pl.when(j < NCHUNK)
                def _():
                    @pl.when(jn < NCHUNK)
                    def _():
                        prefetch(jn, 1 - par)
                    wait4(j, par)
                    compute_chunk(j, par)
            return 0

        lax.fori_loop(0, (KMAX + 2) // 2, super_body, 0)
        plsc.subcore_barrier()
        drain_agg(0)
        zero_agg()
        plsc.subcore_barrier()

        def prefetch1(j, b):
            srcb, tgtb, gb, ptb, ebb, cob, sem = bufs[b]
            base = j * C
            pltpu.sync_copy(tgt_hbm.at[pl.ds(base, C)], tgtb)
            pltpu.async_copy(spill_hbm.at[pl.ds(base, C)], ebb, sem)

        def fix_chunk(j, b):
            srcb, tgtb, gb, ptb, ebb, cob, sem = bufs[b]
            base = j * C
            pltpu.make_async_copy(
                spill_hbm.at[pl.ds(base, C)], ebb, sem).wait()
            clamp_scatter(tgtb, AGG_R, ebb)

        @pl.when(j0 < NCHUNK)
        def _():
            prefetch1(j0, 0)

        def super_fix(m, _):
            for par in range(2):
                k = 2 * m + par
                j = wid + NW * k
                jn = j + NW

                @pl.when(j < NCHUNK)
                def _():
                    @pl.when(jn < NCHUNK)
                    def _():
                        prefetch1(jn, 1 - par)
                    fix_chunk(j, par)
            return 0

        lax.fori_loop(0, (KMAX + 2) // 2, super_fix, 0)
        plsc.subcore_barrier()
        drain_agg(AGG_R)

    return edge_kernel


def _make_embed_gather():
    mesh = plsc.VectorSubcoreMesh(core_axis_name="c", subcore_axis_name="s",
                                  num_cores=NC, num_subcores=NS)

    @functools.partial(
        pl.kernel,
        out_type=jax.ShapeDtypeStruct((2 * 128, 2 * D), jnp.float32),
        mesh=mesh,
        scratch_types=[
            pltpu.VMEM((128,), jnp.int32),
            pltpu.VMEM((128, 2 * D), jnp.float32),
            pltpu.SemaphoreType.DMA,
        ],
    )
    def gather_kernel(total_hbm, idx_hbm, out_hbm, idx_v, rows_v, sem):
        wid = lax.axis_index("s") * NC + lax.axis_index("c")

        @pl.when(wid < 2)
        def _():
            base = wid * 128
            pltpu.sync_copy(idx_hbm.at[pl.ds(base, 128)], idx_v)
            pltpu.async_copy(total_hbm.at[idx_v], rows_v, sem).wait()
            pltpu.sync_copy(rows_v, out_hbm.at[pl.ds(base, 128)])

    return gather_kernel


_get_edge_sc = functools.lru_cache(maxsize=None)(_make_edge_sc)
_get_embed_gather = functools.lru_cache(maxsize=None)(_make_embed_gather)


def _node_mm(h, wcat):
    return pl.pallas_call(
        _node_mm_body,
        grid=(N // _BN,),
        in_specs=[
            pl.BlockSpec((_BN, D), lambda i: (i, 0)),
            pl.BlockSpec((D, 7 * D), lambda i: (0, 0)),
        ],
        out_specs=[
            pl.BlockSpec((_BN, 5 * D), lambda i: (i, 0)),
            pl.BlockSpec((_BN, D), lambda i: (i, 0)),
            pl.BlockSpec((_BN, D), lambda i: (i, 0)),
        ],
        out_shape=[
            jax.ShapeDtypeStruct((N, 5 * D), jnp.float32),
            jax.ShapeDtypeStruct((N, D), jnp.float32),
            jax.ShapeDtypeStruct((N, D), jnp.float32),
        ],
    )(h, wcat)


def _edge_feat(re, tr, rel2d, wre, wtr, ab, comp_cat):
    return pl.pallas_call(
        _edge_feat_body,
        grid=(E // _BE,),
        in_specs=[
            pl.BlockSpec((_BE, A_DIM), lambda i: (i, 0)),
            pl.BlockSpec((_BE, A_DIM), lambda i: (i, 0)),
            pl.BlockSpec((_BE, 1), lambda i: (i, 0)),
            pl.BlockSpec((A_DIM, 2 * D), lambda i: (0, 0)),
            pl.BlockSpec((A_DIM, 2 * D), lambda i: (0, 0)),
            pl.BlockSpec((1, 2 * D), lambda i: (0, 0)),
            pl.BlockSpec((R, 8), lambda i: (0, 0)),
        ],
        out_specs=[
            pl.BlockSpec((_BE, 2 * D), lambda i: (i, 0)),
            pl.BlockSpec((_BE, 4), lambda i: (i, 0)),
            pl.BlockSpec((_BE, 4), lambda i: (i, 0)),
        ],
        out_shape=[
            jax.ShapeDtypeStruct((E, 2 * D), jnp.float32),
            jax.ShapeDtypeStruct((E, 4), jnp.float32),
            jax.ShapeDtypeStruct((E, 4), jnp.float32),
        ],
    )(re, tr, rel2d, wre, wtr, ab, comp_cat)


def _combine(curr, parts):
    return pl.pallas_call(
        _combine_body,
        grid=(N // _BN,),
        in_specs=[
            pl.BlockSpec((_BN, D), lambda i: (i, 0)),
            pl.BlockSpec((1, _BN, D), lambda i: (0, i, 0)),
            pl.BlockSpec((1, _BN, D), lambda i: (1, i, 0)),
        ],
        out_specs=pl.BlockSpec((_BN, D), lambda i: (i, 0)),
        out_shape=jax.ShapeDtypeStruct((N, D), jnp.float32),
    )(curr, parts, parts)


def _final(h1, h2):
    return pl.pallas_call(
        _final_body,
        grid=(N // _BN,),
        in_specs=[pl.BlockSpec((_BN, D), lambda i: (i, 0))] * 2,
        out_specs=pl.BlockSpec((_BN, 2 * D), lambda i: (i, 0)),
        out_shape=jax.ShapeDtypeStruct((N, 2 * D), jnp.float32),
    )(h1, h2)


def _pool(gszf, total):
    return pl.pallas_call(
        _pool_body,
        out_shape=jax.ShapeDtypeStruct((G, 2 * D), jnp.float32),
    )(gszf, total)


def kernel(node_feat, total_edge, total_relation, total_relation_embed,
           total_target_relation, source_node, target_node, graph_sizes,
           params):
    (Wb1, comp1, sloop1, Aw1, Ab1, Bw1, Bb1) = params[0]
    (Wb2, comp2, sloop2, Aw2, Ab2, Bw2, Bb2) = params[1]

    def build_wcat(Wb, sloop, Aw):
        wb_flat = jnp.transpose(Wb, (1, 0, 2)).reshape(D, NB * D)
        return jnp.concatenate(
            [wb_flat, Aw[:, :D].T, sloop, Aw[:, D:2 * D].T], axis=1)

    wcat = jnp.stack([build_wcat(Wb1, sloop1, Aw1),
                      build_wcat(Wb2, sloop2, Aw2)])
    wre = jnp.concatenate(
        [Aw1[:, 2 * D:2 * D + A_DIM].T, Aw2[:, 2 * D:2 * D + A_DIM].T], axis=1)
    wtr = jnp.concatenate(
        [Aw1[:, 2 * D + A_DIM:].T, Aw2[:, 2 * D + A_DIM:].T], axis=1)
    ab = jnp.concatenate([Ab1, Ab2])[None, :]
    comp_cat = jnp.concatenate([comp1, comp2], axis=1)
    bwbb = jnp.stack([
        jnp.concatenate([Bw1[0], jnp.full((16,), Bb1[0] / 16.0)]),
        jnp.concatenate([Bw2[0], jnp.full((16,), Bb2[0] / 16.0)]),
    ])
    lidx = jnp.stack([jnp.zeros((16,), jnp.int32), jnp.ones((16,), jnp.int32)])

    src = total_edge[0].astype(jnp.int32)
    tgt = total_edge[1].astype(jnp.int32)
    rel2d = total_relation.astype(jnp.int32).reshape(E, 1)
    idx_pad = jnp.concatenate([
        jnp.pad(source_node.astype(jnp.int32), (0, 128 - G)),
        jnp.pad(target_node.astype(jnp.int32), (0, 128 - G)),
    ])
    gszf = graph_sizes.astype(jnp.float32).reshape(G, 1)

    eb_cat, co1, co2 = _edge_feat(total_relation_embed, total_target_relation,
                                  rel2d, wre, wtr, ab, comp_cat)
    co_s = jnp.stack([co1.reshape(E * 4), co2.reshape(E * 4)])
    edge_sc = _get_edge_sc()

    def layer_body(h, xs):
        wcat_l, co_l, bwbb_l, lidx_l = xs
        tsrc, curr, ptgt = _node_mm(h, wcat_l)
        parts, _ = edge_sc(tsrc, ptgt, eb_cat, co_l, src, tgt, bwbb_l, lidx_l)
        h_next = _combine(curr, parts)
        return h_next, h_next

    _, hs = lax.scan(layer_body, node_feat, (wcat, co_s, bwbb, lidx))

    total = _final(hs[0], hs[1])
    graph_embed = _pool(gszf, total)
    sg = _get_embed_gather()(total, idx_pad)
    return (graph_embed, sg[:G], sg[128:128 + G])
